# Initial kernel scaffold; baseline (speedup 1.0000x reference)
import jax
import jax.numpy as jnp
from jax.experimental import pallas as pl


def kernel(x, edge_index, edge_attr, W0_0, W1_0, W2_0, Wa_0, W0_1, W1_1, W2_1, Wa_1):
    raise NotImplementedError("write your pallas kernel here")


# speedup vs baseline: 15.3575x; 15.3575x over previous
import functools

import jax
import jax.numpy as jnp
from jax import lax
from jax.experimental import pallas as pl
from jax.experimental.pallas import tpu as pltpu
from jax.experimental.pallas import tpu_sc as plsc

_N = 10000
_E = 320000
_D = 128
_NC = 2
_NS = 16
_NW = _NC * _NS
_EPW = _E // _NW
_K = 80
_NCH = _EPW // _K
_NP = 10240
_NROW = _NP // _NS
_ZCH = 80

_MESH = plsc.VectorSubcoreMesh(core_axis_name="c", subcore_axis_name="s")
_SC_PARAMS = pltpu.CompilerParams(needs_layout_passes=False)


def _f16(val, dtype=jnp.float32):
    return jnp.full((16,), val, dtype)


_GDN = lax.GatherDimensionNumbers(
    offset_dims=(), collapsed_slice_dims=(0,), start_index_map=(0,))


def _vtake(vec, idx):
    return lax.gather(vec, idx[:, None], _GDN, (1,),
                      mode=lax.GatherScatterMode.PROMISE_IN_BOUNDS)


_BN = 2000


def _tc_pre_body(h_ref, w1t_ref, w2t_ref, wa2_ref, z_ref, zi_ref, a2_ref):
    h = h_ref[...]
    z = jnp.dot(h, w1t_ref[...], preferred_element_type=jnp.float32)
    z_ref[...] = z
    zi_ref[...] = jnp.dot(h, w2t_ref[...], preferred_element_type=jnp.float32)
    a2_ref[...] = jnp.dot(z, wa2_ref[...], preferred_element_type=jnp.float32)


def _tc_pre(h, w1t, w2t, wa2):
    return pl.pallas_call(
        _tc_pre_body,
        grid=(_N // _BN,),
        in_specs=[
            pl.BlockSpec((_BN, _D), lambda i: (i, 0)),
            pl.BlockSpec((_D, _D), lambda i: (0, 0)),
            pl.BlockSpec((_D, _D), lambda i: (0, 0)),
            pl.BlockSpec((_D, 2), lambda i: (0, 0)),
        ],
        out_specs=[
            pl.BlockSpec((_BN, _D), lambda i: (i, 0)),
            pl.BlockSpec((_BN, _D), lambda i: (i, 0)),
            pl.BlockSpec((_BN, 2), lambda i: (i, 0)),
        ],
        out_shape=[
            jax.ShapeDtypeStruct((_N, _D), jnp.float32),
            jax.ShapeDtypeStruct((_N, _D), jnp.float32),
            jax.ShapeDtypeStruct((_N, 2), jnp.float32),
        ],
    )(h, w1t, w2t, wa2)


def _tc_max_body(p_ref, o_ref):
    o_ref[...] = jnp.max(p_ref[...], axis=0, keepdims=True)


def _tc_max(parts):
    return pl.pallas_call(
        _tc_max_body,
        out_shape=jax.ShapeDtypeStruct((1, _N), jnp.float32),
    )(parts)


def _tc_invsum_body(p_ref, o_ref):
    o_ref[...] = 1.0 / jnp.sum(p_ref[...], axis=0, keepdims=True)


def _tc_invsum(parts):
    return pl.pallas_call(
        _tc_invsum_body,
        out_shape=jax.ShapeDtypeStruct((1, _N), jnp.float32),
    )(parts)


def _tc_out_body(zi_ref, p0_ref, p1_ref, o_ref):
    o_ref[...] = jnp.maximum(zi_ref[...] + p0_ref[...] + p1_ref[...], 0.0)


def _tc_out(zi, p0, p1):
    return pl.pallas_call(
        _tc_out_body,
        grid=(_N // _BN,),
        in_specs=[
            pl.BlockSpec((_BN, _D), lambda i: (i, 0)),
            pl.BlockSpec((_BN, _D), lambda i: (i, 0)),
            pl.BlockSpec((_BN, _D), lambda i: (i, 0)),
        ],
        out_specs=pl.BlockSpec((_BN, _D), lambda i: (i, 0)),
        out_shape=jax.ShapeDtypeStruct((_N, _D), jnp.float32),
    )(zi, p0, p1)


def _sc_logit_body(src_hbm, dst_hbm, ea_hbm, asrc_hbm, adst_hbm, q_hbm,
                   e_hbm, mpart_hbm,
                   asrc_v, adst_v, m_v, src_v, dst_v, ea_v, e_v, q_v):
    cid = lax.axis_index("c")
    sid = lax.axis_index("s")
    wid = sid * _NC + cid
    base = wid * _EPW
    pltpu.sync_copy(asrc_hbm, asrc_v)
    pltpu.sync_copy(adst_hbm, adst_v)
    pltpu.sync_copy(src_hbm.at[pl.ds(base, _EPW)], src_v)
    pltpu.sync_copy(dst_hbm.at[pl.ds(base, _EPW)], dst_v)
    pltpu.sync_copy(ea_hbm.at[pl.ds(base, _EPW)], ea_v)
    pltpu.sync_copy(q_hbm, q_v)

    neg_inf = _f16(-jnp.inf)

    def init_body(i, carry):
        m_v[pl.ds(i * 16, 16)] = neg_inf
        return carry

    lax.fori_loop(0, _N // 16, init_body, 0)

    q = q_v[...]

    def edge_body(i, carry):
        sl = pl.ds(i * 16, 16)
        s = src_v[sl]
        dvec = dst_v[sl]
        u = (plsc.load_gather(asrc_v, [s]) + plsc.load_gather(adst_v, [dvec])
             + q * ea_v[sl])
        e = jnp.where(u >= 0.0, u, u * 0.01)
        e_v[sl] = e

        def wcond(go):
            return go

        def wbody(go):
            mold = plsc.load_gather(m_v, [dvec])
            plsc.store_scatter(m_v, [dvec], e, mask=e > mold)
            mchk = plsc.load_gather(m_v, [dvec])
            return jnp.any(e > mchk)

        lax.while_loop(wcond, wbody, True)
        return carry

    lax.fori_loop(0, _EPW // 16, edge_body, 0)

    pltpu.sync_copy(e_v, e_hbm.at[pl.ds(base, _EPW)])
    pltpu.sync_copy(m_v, mpart_hbm.at[wid])


_sc_logit = functools.partial(
    pl.kernel,
    out_type=[jax.ShapeDtypeStruct((_E,), jnp.float32),
              jax.ShapeDtypeStruct((_NW, _N), jnp.float32)],
    mesh=_MESH,
    compiler_params=_SC_PARAMS,
    scratch_types=[
        pltpu.VMEM((_N,), jnp.float32),
        pltpu.VMEM((_N,), jnp.float32),
        pltpu.VMEM((_N,), jnp.float32),
        pltpu.VMEM((_EPW,), jnp.int32),
        pltpu.VMEM((_EPW,), jnp.int32),
        pltpu.VMEM((_EPW,), jnp.float32),
        pltpu.VMEM((_EPW,), jnp.float32),
        pltpu.VMEM((16,), jnp.float32),
    ],
)(_sc_logit_body)


def _sc_expsum_body(dst_hbm, e_hbm, m_hbm,
                    ee_hbm, spart_hbm,
                    m_v, s_v, dst_v, e_v, ee_v):
    cid = lax.axis_index("c")
    sid = lax.axis_index("s")
    wid = sid * _NC + cid
    base = wid * _EPW
    pltpu.sync_copy(m_hbm, m_v)
    pltpu.sync_copy(dst_hbm.at[pl.ds(base, _EPW)], dst_v)
    pltpu.sync_copy(e_hbm.at[pl.ds(base, _EPW)], e_v)

    zero = _f16(0.0)

    def init_body(i, carry):
        s_v[pl.ds(i * 16, 16)] = zero
        return carry

    lax.fori_loop(0, _N // 16, init_body, 0)

    iota = lax.iota(jnp.int32, 16)

    def edge_body(i, carry):
        sl = pl.ds(i * 16, 16)
        dvec = dst_v[sl]
        ee = jnp.exp(e_v[sl] - plsc.load_gather(m_v, [dvec]))
        ee_v[sl] = ee
        k_s, v_s = plsc.sort_key_val(dvec, ee)
        cs = plsc.cumsum(v_s)
        nxt = _vtake(k_s, jnp.minimum(iota + 1, 15))
        is_last = (k_s != nxt) | (iota == 15)
        is_first = k_s != _vtake(k_s, jnp.maximum(iota - 1, 0))
        is_first = is_first | (iota == 0)
        start = plsc.cummax(jnp.where(is_first, iota, 0))
        cs_prev = jnp.where(
            start == 0, 0.0, _vtake(cs, jnp.maximum(start - 1, 0)))
        seg_tot = cs - cs_prev
        old = plsc.load_gather(s_v, [k_s])
        plsc.store_scatter(s_v, [k_s], old + seg_tot, mask=is_last)
        return carry

    lax.fori_loop(0, _EPW // 16, edge_body, 0)

    pltpu.sync_copy(ee_v, ee_hbm.at[pl.ds(base, _EPW)])
    pltpu.sync_copy(s_v, spart_hbm.at[wid])


_sc_expsum = functools.partial(
    pl.kernel,
    out_type=[jax.ShapeDtypeStruct((_E,), jnp.float32),
              jax.ShapeDtypeStruct((_NW, _N), jnp.float32)],
    mesh=_MESH,
    compiler_params=_SC_PARAMS,
    scratch_types=[
        pltpu.VMEM((_N,), jnp.float32),
        pltpu.VMEM((_N,), jnp.float32),
        pltpu.VMEM((_EPW,), jnp.int32),
        pltpu.VMEM((_EPW,), jnp.float32),
        pltpu.VMEM((_EPW,), jnp.float32),
    ],
)(_sc_expsum_body)


def _sc_agg_body(src3_hbm, dst3_hbm, alpha_hbm, z_hbm,
                 out_hbm,
                 src2_v, dst2_v, alpha_v, rows_v, acc, gsem):
    cid = lax.axis_index("c")
    sid = lax.axis_index("s")
    wid = sid * _NC + cid
    pltpu.sync_copy(src3_hbm.at[wid], src2_v)
    pltpu.sync_copy(dst3_hbm.at[wid], dst2_v)

    for r in range(_K):
        for j in range(_D // 16):
            rows_v[r, pl.ds(j * 16, 16)] = _f16(0.0)

    def zero_body(r, carry):
        pltpu.sync_copy(rows_v, acc.at[pl.ds(sid * _NROW + r * _ZCH, _ZCH)])
        return carry

    lax.fori_loop(0, _NROW // _ZCH, zero_body, 0)
    plsc.subcore_barrier()

    def chunk_body(c, carry):
        pltpu.async_copy(z_hbm.at[src2_v.at[c]], rows_v, gsem).wait()
        pltpu.sync_copy(alpha_hbm.at[pl.ds(wid * _EPW + c * _K, _K)], alpha_v)

        def scale_body(k, carry2):
            a16 = plsc.load_gather(alpha_v, [_f16(k, jnp.int32)])
            for j in range(_D // 16):
                sl = pl.ds(j * 16, 16)
                rows_v[k, sl] = rows_v[k, sl] * a16
            return carry2

        lax.fori_loop(0, _K, scale_body, 0)
        pltpu.sync_copy(rows_v, acc.at[dst2_v.at[c]], add=True)
        return carry

    lax.fori_loop(0, _NCH, chunk_body, 0)
    plsc.subcore_barrier()

    def out_body(r, carry):
        sl = pl.ds(sid * _NROW + r * _ZCH, _ZCH)
        pltpu.sync_copy(acc.at[sl], out_hbm.at[cid].at[sl])
        return carry

    lax.fori_loop(0, _NROW // _ZCH, out_body, 0)


_sc_agg = functools.partial(
    pl.kernel,
    out_type=jax.ShapeDtypeStruct((_NC, _NP, _D), jnp.float32),
    mesh=_MESH,
    compiler_params=_SC_PARAMS,
    scratch_types=[
        pltpu.VMEM((_NCH, _K), jnp.int32),
        pltpu.VMEM((_NCH, _K), jnp.int32),
        pltpu.VMEM((_K,), jnp.float32),
        pltpu.VMEM((_K, _D), jnp.float32),
        pltpu.VMEM_SHARED((_NP, _D), jnp.float32),
        pltpu.SemaphoreType.DMA,
    ],
)(_sc_agg_body)


def _sc_alpha_body(dst_hbm, ee_hbm, invd_hbm, alpha_hbm,
                   invd_v, dst_v, ee_v, alpha_v):
    cid = lax.axis_index("c")
    sid = lax.axis_index("s")
    wid = sid * _NC + cid
    base = wid * _EPW
    pltpu.sync_copy(invd_hbm, invd_v)
    pltpu.sync_copy(dst_hbm.at[pl.ds(base, _EPW)], dst_v)
    pltpu.sync_copy(ee_hbm.at[pl.ds(base, _EPW)], ee_v)

    def edge_body(i, carry):
        sl = pl.ds(i * 16, 16)
        alpha_v[sl] = ee_v[sl] * plsc.load_gather(invd_v, [dst_v[sl]])
        return carry

    lax.fori_loop(0, _EPW // 16, edge_body, 0)
    pltpu.sync_copy(alpha_v, alpha_hbm.at[pl.ds(base, _EPW)])


_sc_alpha = functools.partial(
    pl.kernel,
    out_type=jax.ShapeDtypeStruct((_E,), jnp.float32),
    mesh=_MESH,
    compiler_params=_SC_PARAMS,
    scratch_types=[
        pltpu.VMEM((_N,), jnp.float32),
        pltpu.VMEM((_EPW,), jnp.int32),
        pltpu.VMEM((_EPW,), jnp.float32),
        pltpu.VMEM((_EPW,), jnp.float32),
    ],
)(_sc_alpha_body)


def _layer(h, src, dst, src3, dst3, ea, W0, W1, W2, Wa):
    w1t = W1.T
    w2t = W2.T
    wa2 = jnp.stack([Wa[0, :_D], Wa[0, _D:2 * _D]], axis=1)
    q16 = jnp.full((16,), W0[0, 0] * Wa[0, 2 * _D], jnp.float32)

    z, zi, a2 = _tc_pre(h, w1t, w2t, wa2)
    asrc = a2[:, 0]
    adst = a2[:, 1]
    e, mpart = _sc_logit(src, dst, ea, asrc, adst, q16)
    m = _tc_max(mpart).reshape(_N)
    ee, spart = _sc_expsum(dst, e, m)
    invd = _tc_invsum(spart).reshape(_N)
    alpha = _sc_alpha(dst, ee, invd)
    parts = _sc_agg(src3, dst3, alpha, z)
    return _tc_out(zi, parts[0, :_N], parts[1, :_N])


def kernel(x, edge_index, edge_attr, W0_0, W1_0, W2_0, Wa_0,
           W0_1, W1_1, W2_1, Wa_1):
    src = edge_index[0]
    dst = edge_index[1]
    ea = edge_attr[:, 0]
    src3 = src.reshape(_NW, _NCH, _K)
    dst3 = dst.reshape(_NW, _NCH, _K)
    h = _layer(x, src, dst, src3, dst3, ea, W0_0, W1_0, W2_0, Wa_0)
    h = _layer(h, src, dst, src3, dst3, ea, W0_1, W1_1, W2_1, Wa_1)
    return h


# speedup vs baseline: 15.7374x; 1.0247x over previous
import functools

import jax
import jax.numpy as jnp
from jax import lax
from jax.experimental import pallas as pl
from jax.experimental.pallas import tpu as pltpu
from jax.experimental.pallas import tpu_sc as plsc

_N = 10000
_E = 320000
_D = 128
_NC = 2
_NS = 16
_NW = _NC * _NS
_EPW = _E // _NW
_K = 80
_NCH = _EPW // _K
_NP = 10240
_NROW = _NP // _NS
_ZCH = 80

_MESH = plsc.VectorSubcoreMesh(core_axis_name="c", subcore_axis_name="s")
_SC_PARAMS = pltpu.CompilerParams(needs_layout_passes=False)


def _f16(val, dtype=jnp.float32):
    return jnp.full((16,), val, dtype)


_GDN = lax.GatherDimensionNumbers(
    offset_dims=(), collapsed_slice_dims=(0,), start_index_map=(0,))


def _vtake(vec, idx):
    return lax.gather(vec, idx[:, None], _GDN, (1,),
                      mode=lax.GatherScatterMode.PROMISE_IN_BOUNDS)


_IOTA16 = None


def _seg_last_and_prefix(keys, vals, op):
    iota = lax.iota(jnp.int32, 16)
    v = vals
    for step in (1, 2, 4, 8):
        idx = jnp.maximum(iota - step, 0)
        same = (_vtake(keys, idx) == keys) & (iota >= step)
        v = jnp.where(same, op(v, _vtake(v, idx)), v)
    nxt = _vtake(keys, jnp.minimum(iota + 1, 15))
    is_last = (keys != nxt) | (iota == 15)
    return v, is_last


_BN = 2000


def _tc_pre_body(h_ref, w1t_ref, w2t_ref, wa2_ref, z_ref, zi_ref, a2_ref):
    h = h_ref[...]
    z = jnp.dot(h, w1t_ref[...], preferred_element_type=jnp.float32)
    z_ref[...] = z
    zi_ref[...] = jnp.dot(h, w2t_ref[...], preferred_element_type=jnp.float32)
    a2_ref[...] = jnp.dot(z, wa2_ref[...], preferred_element_type=jnp.float32)


def _tc_pre(h, w1t, w2t, wa2):
    return pl.pallas_call(
        _tc_pre_body,
        grid=(_N // _BN,),
        in_specs=[
            pl.BlockSpec((_BN, _D), lambda i: (i, 0)),
            pl.BlockSpec((_D, _D), lambda i: (0, 0)),
            pl.BlockSpec((_D, _D), lambda i: (0, 0)),
            pl.BlockSpec((_D, 2), lambda i: (0, 0)),
        ],
        out_specs=[
            pl.BlockSpec((_BN, _D), lambda i: (i, 0)),
            pl.BlockSpec((_BN, _D), lambda i: (i, 0)),
            pl.BlockSpec((_BN, 2), lambda i: (i, 0)),
        ],
        out_shape=[
            jax.ShapeDtypeStruct((_N, _D), jnp.float32),
            jax.ShapeDtypeStruct((_N, _D), jnp.float32),
            jax.ShapeDtypeStruct((_N, 2), jnp.float32),
        ],
    )(h, w1t, w2t, wa2)


def _tc_max_body(p_ref, o_ref):
    o_ref[...] = jnp.max(p_ref[...], axis=0, keepdims=True)


def _tc_max(parts):
    return pl.pallas_call(
        _tc_max_body,
        out_shape=jax.ShapeDtypeStruct((1, _N), jnp.float32),
    )(parts)


def _tc_invsum_body(p_ref, o_ref):
    o_ref[...] = 1.0 / jnp.sum(p_ref[...], axis=0, keepdims=True)


def _tc_invsum(parts):
    return pl.pallas_call(
        _tc_invsum_body,
        out_shape=jax.ShapeDtypeStruct((1, _N), jnp.float32),
    )(parts)


def _tc_out_body(zi_ref, p0_ref, p1_ref, o_ref):
    o_ref[...] = jnp.maximum(zi_ref[...] + p0_ref[...] + p1_ref[...], 0.0)


def _tc_out(zi, p0, p1):
    return pl.pallas_call(
        _tc_out_body,
        grid=(_N // _BN,),
        in_specs=[
            pl.BlockSpec((_BN, _D), lambda i: (i, 0)),
            pl.BlockSpec((_BN, _D), lambda i: (i, 0)),
            pl.BlockSpec((_BN, _D), lambda i: (i, 0)),
        ],
        out_specs=pl.BlockSpec((_BN, _D), lambda i: (i, 0)),
        out_shape=jax.ShapeDtypeStruct((_N, _D), jnp.float32),
    )(zi, p0, p1)


def _sc_logit_body(src_hbm, dst_hbm, ea_hbm, asrc_hbm, adst_hbm, q_hbm,
                   e_hbm, mpart_hbm,
                   asrc_v, adst_v, m_v, src_v, dst_v, ea_v, e_v, q_v):
    cid = lax.axis_index("c")
    sid = lax.axis_index("s")
    wid = sid * _NC + cid
    base = wid * _EPW
    pltpu.sync_copy(asrc_hbm, asrc_v)
    pltpu.sync_copy(adst_hbm, adst_v)
    pltpu.sync_copy(src_hbm.at[pl.ds(base, _EPW)], src_v)
    pltpu.sync_copy(dst_hbm.at[pl.ds(base, _EPW)], dst_v)
    pltpu.sync_copy(ea_hbm.at[pl.ds(base, _EPW)], ea_v)
    pltpu.sync_copy(q_hbm, q_v)

    neg_inf = _f16(-jnp.inf)

    def init_body(i, carry):
        for u in range(5):
            m_v[pl.ds((i * 5 + u) * 16, 16)] = neg_inf
        return carry

    lax.fori_loop(0, _N // 80, init_body, 0)

    q = q_v[...]

    def edge_body(i, carry):
        for u in range(5):
            sl = pl.ds((i * 5 + u) * 16, 16)
            s = src_v[sl]
            dvec = dst_v[sl]
            u_ = (plsc.load_gather(asrc_v, [s])
                  + plsc.load_gather(adst_v, [dvec]) + q * ea_v[sl])
            e = jnp.where(u_ >= 0.0, u_, u_ * 0.01)
            e_v[sl] = e
            k_s, v_s = plsc.sort_key_val(dvec, e)
            segmax, is_last = _seg_last_and_prefix(k_s, v_s, jnp.maximum)
            old = plsc.load_gather(m_v, [k_s])
            plsc.store_scatter(m_v, [k_s], jnp.maximum(old, segmax),
                               mask=is_last)
        return carry

    lax.fori_loop(0, _EPW // 80, edge_body, 0)

    pltpu.sync_copy(e_v, e_hbm.at[pl.ds(base, _EPW)])
    pltpu.sync_copy(m_v, mpart_hbm.at[wid])


_sc_logit = functools.partial(
    pl.kernel,
    out_type=[jax.ShapeDtypeStruct((_E,), jnp.float32),
              jax.ShapeDtypeStruct((_NW, _N), jnp.float32)],
    mesh=_MESH,
    compiler_params=_SC_PARAMS,
    scratch_types=[
        pltpu.VMEM((_N,), jnp.float32),
        pltpu.VMEM((_N,), jnp.float32),
        pltpu.VMEM((_N,), jnp.float32),
        pltpu.VMEM((_EPW,), jnp.int32),
        pltpu.VMEM((_EPW,), jnp.int32),
        pltpu.VMEM((_EPW,), jnp.float32),
        pltpu.VMEM((_EPW,), jnp.float32),
        pltpu.VMEM((16,), jnp.float32),
    ],
)(_sc_logit_body)


def _sc_expsum_body(dst_hbm, e_hbm, m_hbm,
                    ee_hbm, spart_hbm,
                    m_v, s_v, dst_v, e_v, ee_v):
    cid = lax.axis_index("c")
    sid = lax.axis_index("s")
    wid = sid * _NC + cid
    base = wid * _EPW
    pltpu.sync_copy(m_hbm, m_v)
    pltpu.sync_copy(dst_hbm.at[pl.ds(base, _EPW)], dst_v)
    pltpu.sync_copy(e_hbm.at[pl.ds(base, _EPW)], e_v)

    zero = _f16(0.0)

    def init_body(i, carry):
        for u in range(5):
            s_v[pl.ds((i * 5 + u) * 16, 16)] = zero
        return carry

    lax.fori_loop(0, _N // 80, init_body, 0)

    def edge_body(i, carry):
        for u in range(5):
            sl = pl.ds((i * 5 + u) * 16, 16)
            dvec = dst_v[sl]
            ee = jnp.exp(e_v[sl] - plsc.load_gather(m_v, [dvec]))
            ee_v[sl] = ee
            k_s, v_s = plsc.sort_key_val(dvec, ee)
            seg_tot, is_last = _seg_last_and_prefix(k_s, v_s, jnp.add)
            old = plsc.load_gather(s_v, [k_s])
            plsc.store_scatter(s_v, [k_s], old + seg_tot, mask=is_last)
        return carry

    lax.fori_loop(0, _EPW // 80, edge_body, 0)

    pltpu.sync_copy(ee_v, ee_hbm.at[pl.ds(base, _EPW)])
    pltpu.sync_copy(s_v, spart_hbm.at[wid])


_sc_expsum = functools.partial(
    pl.kernel,
    out_type=[jax.ShapeDtypeStruct((_E,), jnp.float32),
              jax.ShapeDtypeStruct((_NW, _N), jnp.float32)],
    mesh=_MESH,
    compiler_params=_SC_PARAMS,
    scratch_types=[
        pltpu.VMEM((_N,), jnp.float32),
        pltpu.VMEM((_N,), jnp.float32),
        pltpu.VMEM((_EPW,), jnp.int32),
        pltpu.VMEM((_EPW,), jnp.float32),
        pltpu.VMEM((_EPW,), jnp.float32),
    ],
)(_sc_expsum_body)


def _sc_agg_body(src3_hbm, dst3_hbm, alpha_hbm, z_hbm,
                 out_hbm,
                 src2_v, dst2_v, alpha_v, rows_v, acc, gsem):
    cid = lax.axis_index("c")
    sid = lax.axis_index("s")
    wid = sid * _NC + cid
    pltpu.sync_copy(src3_hbm.at[wid], src2_v)
    pltpu.sync_copy(dst3_hbm.at[wid], dst2_v)

    for r in range(_K):
        for j in range(_D // 16):
            rows_v[r, pl.ds(j * 16, 16)] = _f16(0.0)

    def zero_body(r, carry):
        pltpu.sync_copy(rows_v, acc.at[pl.ds(sid * _NROW + r * _ZCH, _ZCH)])
        return carry

    lax.fori_loop(0, _NROW // _ZCH, zero_body, 0)
    plsc.subcore_barrier()

    def chunk_body(c, carry):
        pltpu.async_copy(z_hbm.at[src2_v.at[c]], rows_v, gsem).wait()
        pltpu.sync_copy(alpha_hbm.at[pl.ds(wid * _EPW + c * _K, _K)], alpha_v)

        def scale_body(k, carry2):
            a16 = plsc.load_gather(alpha_v, [_f16(k, jnp.int32)])
            for j in range(_D // 16):
                sl = pl.ds(j * 16, 16)
                rows_v[k, sl] = rows_v[k, sl] * a16
            return carry2

        lax.fori_loop(0, _K, scale_body, 0)
        pltpu.sync_copy(rows_v, acc.at[dst2_v.at[c]], add=True)
        return carry

    lax.fori_loop(0, _NCH, chunk_body, 0)
    plsc.subcore_barrier()

    def out_body(r, carry):
        sl = pl.ds(sid * _NROW + r * _ZCH, _ZCH)
        pltpu.sync_copy(acc.at[sl], out_hbm.at[cid].at[sl])
        return carry

    lax.fori_loop(0, _NROW // _ZCH, out_body, 0)


_sc_agg = functools.partial(
    pl.kernel,
    out_type=jax.ShapeDtypeStruct((_NC, _NP, _D), jnp.float32),
    mesh=_MESH,
    compiler_params=_SC_PARAMS,
    scratch_types=[
        pltpu.VMEM((_NCH, _K), jnp.int32),
        pltpu.VMEM((_NCH, _K), jnp.int32),
        pltpu.VMEM((_K,), jnp.float32),
        pltpu.VMEM((_K, _D), jnp.float32),
        pltpu.VMEM_SHARED((_NP, _D), jnp.float32),
        pltpu.SemaphoreType.DMA,
    ],
)(_sc_agg_body)


def _sc_alpha_body(dst_hbm, ee_hbm, invd_hbm, alpha_hbm,
                   invd_v, dst_v, ee_v, alpha_v):
    cid = lax.axis_index("c")
    sid = lax.axis_index("s")
    wid = sid * _NC + cid
    base = wid * _EPW
    pltpu.sync_copy(invd_hbm, invd_v)
    pltpu.sync_copy(dst_hbm.at[pl.ds(base, _EPW)], dst_v)
    pltpu.sync_copy(ee_hbm.at[pl.ds(base, _EPW)], ee_v)

    def edge_body(i, carry):
        for u in range(5):
            sl = pl.ds((i * 5 + u) * 16, 16)
            alpha_v[sl] = ee_v[sl] * plsc.load_gather(invd_v, [dst_v[sl]])
        return carry

    lax.fori_loop(0, _EPW // 80, edge_body, 0)
    pltpu.sync_copy(alpha_v, alpha_hbm.at[pl.ds(base, _EPW)])


_sc_alpha = functools.partial(
    pl.kernel,
    out_type=jax.ShapeDtypeStruct((_E,), jnp.float32),
    mesh=_MESH,
    compiler_params=_SC_PARAMS,
    scratch_types=[
        pltpu.VMEM((_N,), jnp.float32),
        pltpu.VMEM((_EPW,), jnp.int32),
        pltpu.VMEM((_EPW,), jnp.float32),
        pltpu.VMEM((_EPW,), jnp.float32),
    ],
)(_sc_alpha_body)


def _layer(h, src, dst, src3, dst3, ea, W0, W1, W2, Wa):
    w1t = W1.T
    w2t = W2.T
    wa2 = jnp.stack([Wa[0, :_D], Wa[0, _D:2 * _D]], axis=1)
    q16 = jnp.full((16,), W0[0, 0] * Wa[0, 2 * _D], jnp.float32)

    z, zi, a2 = _tc_pre(h, w1t, w2t, wa2)
    asrc = a2[:, 0]
    adst = a2[:, 1]
    e, mpart = _sc_logit(src, dst, ea, asrc, adst, q16)
    m = _tc_max(mpart).reshape(_N)
    ee, spart = _sc_expsum(dst, e, m)
    invd = _tc_invsum(spart).reshape(_N)
    alpha = _sc_alpha(dst, ee, invd)
    parts = _sc_agg(src3, dst3, alpha, z)
    return _tc_out(zi, parts[0, :_N], parts[1, :_N])


def kernel(x, edge_index, edge_attr, W0_0, W1_0, W2_0, Wa_0,
           W0_1, W1_1, W2_1, Wa_1):
    src = edge_index[0]
    dst = edge_index[1]
    ea = edge_attr[:, 0]
    src3 = src.reshape(_NW, _NCH, _K)
    dst3 = dst.reshape(_NW, _NCH, _K)
    h = _layer(x, src, dst, src3, dst3, ea, W0_0, W1_0, W2_0, Wa_0)
    h = _layer(h, src, dst, src3, dst3, ea, W0_1, W1_1, W2_1, Wa_1)
    return h


# speedup vs baseline: 22.2743x; 1.4154x over previous
import functools

import jax
import jax.numpy as jnp
from jax import lax
from jax.experimental import pallas as pl
from jax.experimental.pallas import tpu as pltpu
from jax.experimental.pallas import tpu_sc as plsc

_N = 10000
_E = 320000
_D = 128
_DH = _D // 2
_NC = 2
_NS = 16
_NW = _NC * _NS
_EPW = _E // _NW
_K = 100
_NCH = _EPW // _K
_NP = 10240
_NROW = _NP // _NS
_ZCH = 80

_MESH = plsc.VectorSubcoreMesh(core_axis_name="c", subcore_axis_name="s")
_SC_PARAMS = pltpu.CompilerParams(needs_layout_passes=False)
_SC_PARAMS_NT = pltpu.CompilerParams(needs_layout_passes=False,
                                     use_tc_tiling_on_sc=False)


def _f16(val, dtype=jnp.float32):
    return jnp.full((16,), val, dtype)


_GDN = lax.GatherDimensionNumbers(
    offset_dims=(), collapsed_slice_dims=(0,), start_index_map=(0,))


def _vtake(vec, idx):
    return lax.gather(vec, idx[:, None], _GDN, (1,),
                      mode=lax.GatherScatterMode.PROMISE_IN_BOUNDS)


def _seg_last_and_prefix(keys, vals, op):
    iota = lax.iota(jnp.int32, 16)
    v = vals
    for step in (1, 2, 4, 8):
        idx = jnp.maximum(iota - step, 0)
        same = (_vtake(keys, idx) == keys) & (iota >= step)
        v = jnp.where(same, op(v, _vtake(v, idx)), v)
    nxt = _vtake(keys, jnp.minimum(iota + 1, 15))
    is_last = (keys != nxt) | (iota == 15)
    return v, is_last


_BN = 2000


def _tc_pre_body(h_ref, w1t_ref, w2t_ref, wa2_ref,
                 z0_ref, z1_ref, zi_ref, a2_ref):
    h = h_ref[...]
    z = jnp.dot(h, w1t_ref[...], preferred_element_type=jnp.float32)
    z0_ref[...] = z[:, :_DH]
    z1_ref[...] = z[:, _DH:]
    zi_ref[...] = jnp.dot(h, w2t_ref[...], preferred_element_type=jnp.float32)
    a2_ref[...] = jnp.dot(z, wa2_ref[...], preferred_element_type=jnp.float32)


def _tc_pre(h, w1t, w2t, wa2):
    return pl.pallas_call(
        _tc_pre_body,
        grid=(_N // _BN,),
        in_specs=[
            pl.BlockSpec((_BN, _D), lambda i: (i, 0)),
            pl.BlockSpec((_D, _D), lambda i: (0, 0)),
            pl.BlockSpec((_D, _D), lambda i: (0, 0)),
            pl.BlockSpec((_D, 2), lambda i: (0, 0)),
        ],
        out_specs=[
            pl.BlockSpec((_BN, _DH), lambda i: (i, 0)),
            pl.BlockSpec((_BN, _DH), lambda i: (i, 0)),
            pl.BlockSpec((_BN, _D), lambda i: (i, 0)),
            pl.BlockSpec((_BN, 2), lambda i: (i, 0)),
        ],
        out_shape=[
            jax.ShapeDtypeStruct((_N, _DH), jnp.float32),
            jax.ShapeDtypeStruct((_N, _DH), jnp.float32),
            jax.ShapeDtypeStruct((_N, _D), jnp.float32),
            jax.ShapeDtypeStruct((_N, 2), jnp.float32),
        ],
    )(h, w1t, w2t, wa2)


def _tc_max_body(p_ref, o_ref):
    o_ref[...] = jnp.max(p_ref[...], axis=0, keepdims=True)


def _tc_max(parts):
    return pl.pallas_call(
        _tc_max_body,
        out_shape=jax.ShapeDtypeStruct((1, _N), jnp.float32),
    )(parts)


def _tc_invsum_body(p_ref, o_ref):
    o_ref[...] = 1.0 / jnp.sum(p_ref[...], axis=0, keepdims=True)


def _tc_invsum(parts):
    return pl.pallas_call(
        _tc_invsum_body,
        out_shape=jax.ShapeDtypeStruct((1, _N), jnp.float32),
    )(parts)


def _tc_out_body(zi_ref, pa0_ref, pa1_ref, pb0_ref, pb1_ref, inv_ref, o_ref):
    zi = zi_ref[...]
    inv = inv_ref[...]
    left = zi[:, :_DH] + inv * (pa0_ref[...] + pa1_ref[...])
    right = zi[:, _DH:] + inv * (pb0_ref[...] + pb1_ref[...])
    o_ref[...] = jnp.maximum(jnp.concatenate([left, right], axis=1), 0.0)


def _tc_out(zi, pa0, pa1, pb0, pb1, inv_col):
    return pl.pallas_call(
        _tc_out_body,
        grid=(_N // _BN,),
        in_specs=[
            pl.BlockSpec((_BN, _D), lambda i: (i, 0)),
            pl.BlockSpec((_BN, _DH), lambda i: (i, 0)),
            pl.BlockSpec((_BN, _DH), lambda i: (i, 0)),
            pl.BlockSpec((_BN, _DH), lambda i: (i, 0)),
            pl.BlockSpec((_BN, _DH), lambda i: (i, 0)),
            pl.BlockSpec((_BN, 1), lambda i: (i, 0)),
        ],
        out_specs=pl.BlockSpec((_BN, _D), lambda i: (i, 0)),
        out_shape=jax.ShapeDtypeStruct((_N, _D), jnp.float32),
    )(zi, pa0, pa1, pb0, pb1, inv_col)


def _sc_logit_body(src_hbm, dst_hbm, ea_hbm, asrc_hbm, adst_hbm, q_hbm,
                   e_hbm, mpart_hbm,
                   asrc_v, adst_v, m_v, src_v, dst_v, ea_v, e_v, q_v):
    cid = lax.axis_index("c")
    sid = lax.axis_index("s")
    wid = sid * _NC + cid
    base = wid * _EPW
    pltpu.sync_copy(asrc_hbm, asrc_v)
    pltpu.sync_copy(adst_hbm, adst_v)
    pltpu.sync_copy(src_hbm.at[pl.ds(base, _EPW)], src_v)
    pltpu.sync_copy(dst_hbm.at[pl.ds(base, _EPW)], dst_v)
    pltpu.sync_copy(ea_hbm.at[pl.ds(base, _EPW)], ea_v)
    pltpu.sync_copy(q_hbm, q_v)

    neg_inf = _f16(-jnp.inf)

    def init_body(i, carry):
        for u in range(5):
            m_v[pl.ds((i * 5 + u) * 16, 16)] = neg_inf
        return carry

    lax.fori_loop(0, _N // 80, init_body, 0)

    q = q_v[...]

    def edge_body(i, carry):
        for u in range(5):
            sl = pl.ds((i * 5 + u) * 16, 16)
            s = src_v[sl]
            dvec = dst_v[sl]
            u_ = (plsc.load_gather(asrc_v, [s])
                  + plsc.load_gather(adst_v, [dvec]) + q * ea_v[sl])
            e = jnp.where(u_ >= 0.0, u_, u_ * 0.01)
            e_v[sl] = e
            k_s, v_s = plsc.sort_key_val(dvec, e)
            segmax, is_last = _seg_last_and_prefix(k_s, v_s, jnp.maximum)
            old = plsc.load_gather(m_v, [k_s])
            plsc.store_scatter(m_v, [k_s], jnp.maximum(old, segmax),
                               mask=is_last)
        return carry

    lax.fori_loop(0, _EPW // 80, edge_body, 0)

    pltpu.sync_copy(e_v, e_hbm.at[pl.ds(base, _EPW)])
    pltpu.sync_copy(m_v, mpart_hbm.at[wid])


_sc_logit = functools.partial(
    pl.kernel,
    out_type=[jax.ShapeDtypeStruct((_E,), jnp.float32),
              jax.ShapeDtypeStruct((_NW, _N), jnp.float32)],
    mesh=_MESH,
    compiler_params=_SC_PARAMS,
    scratch_types=[
        pltpu.VMEM((_N,), jnp.float32),
        pltpu.VMEM((_N,), jnp.float32),
        pltpu.VMEM((_N,), jnp.float32),
        pltpu.VMEM((_EPW,), jnp.int32),
        pltpu.VMEM((_EPW,), jnp.int32),
        pltpu.VMEM((_EPW,), jnp.float32),
        pltpu.VMEM((_EPW,), jnp.float32),
        pltpu.VMEM((16,), jnp.float32),
    ],
)(_sc_logit_body)


def _sc_expsum_body(dst_hbm, e_hbm, m_hbm,
                    ee_hbm, spart_hbm,
                    m_v, s_v, dst_v, e_v, ee_v):
    cid = lax.axis_index("c")
    sid = lax.axis_index("s")
    wid = sid * _NC + cid
    base = wid * _EPW
    pltpu.sync_copy(m_hbm, m_v)
    pltpu.sync_copy(dst_hbm.at[pl.ds(base, _EPW)], dst_v)
    pltpu.sync_copy(e_hbm.at[pl.ds(base, _EPW)], e_v)

    zero = _f16(0.0)

    def init_body(i, carry):
        for u in range(5):
            s_v[pl.ds((i * 5 + u) * 16, 16)] = zero
        return carry

    lax.fori_loop(0, _N // 80, init_body, 0)

    def edge_body(i, carry):
        for u in range(5):
            sl = pl.ds((i * 5 + u) * 16, 16)
            dvec = dst_v[sl]
            ee = jnp.exp(e_v[sl] - plsc.load_gather(m_v, [dvec]))
            ee_v[sl] = ee
            k_s, v_s = plsc.sort_key_val(dvec, ee)
            seg_tot, is_last = _seg_last_and_prefix(k_s, v_s, jnp.add)
            old = plsc.load_gather(s_v, [k_s])
            plsc.store_scatter(s_v, [k_s], old + seg_tot, mask=is_last)
        return carry

    lax.fori_loop(0, _EPW // 80, edge_body, 0)

    pltpu.sync_copy(ee_v, ee_hbm.at[pl.ds(base, _EPW)])
    pltpu.sync_copy(s_v, spart_hbm.at[wid])


_sc_expsum = functools.partial(
    pl.kernel,
    out_type=[jax.ShapeDtypeStruct((_E,), jnp.float32),
              jax.ShapeDtypeStruct((_NW, _N), jnp.float32)],
    mesh=_MESH,
    compiler_params=_SC_PARAMS,
    scratch_types=[
        pltpu.VMEM((_N,), jnp.float32),
        pltpu.VMEM((_N,), jnp.float32),
        pltpu.VMEM((_EPW,), jnp.int32),
        pltpu.VMEM((_EPW,), jnp.float32),
        pltpu.VMEM((_EPW,), jnp.float32),
    ],
)(_sc_expsum_body)


def _sc_agg_body(src3_hbm, dst3_hbm, ee_hbm, z_hbm,
                 out_hbm,
                 src2_v, dst2_v, ee_v, r0, r1, r2, r3, acc,
                 g0, g1, g2, g3, s0, s1, s2, s3):
    rows = (r0, r1, r2, r3)
    gsems = (g0, g1, g2, g3)
    ssems = (s0, s1, s2, s3)
    cid = lax.axis_index("c")
    sid = lax.axis_index("s")
    wid = sid * _NC + cid
    pltpu.sync_copy(src3_hbm.at[wid], src2_v)
    pltpu.sync_copy(dst3_hbm.at[wid], dst2_v)
    pltpu.sync_copy(ee_hbm.at[pl.ds(wid * _EPW, _EPW)], ee_v)

    for r in range(_ZCH):
        for j in range(_DH // 16):
            r0[r, pl.ds(j * 16, 16)] = _f16(0.0)

    def zero_body(r, carry):
        pltpu.sync_copy(r0.at[pl.ds(0, _ZCH)],
                        acc.at[pl.ds(sid * _NROW + r * _ZCH, _ZCH)])
        return carry

    lax.fori_loop(0, _NROW // _ZCH, zero_body, 0)
    plsc.subcore_barrier()

    def start_gather(c, b):
        pltpu.async_copy(z_hbm.at[src2_v.at[c]], rows[b], gsems[b])

    def wait_gather(c, b):
        pltpu.make_async_copy(z_hbm.at[src2_v.at[c]], rows[b],
                              gsems[b]).wait()

    def start_scatter(c, b):
        pltpu.async_copy(rows[b], acc.at[dst2_v.at[c]], ssems[b], add=True)

    def wait_scatter(c, b):
        pltpu.make_async_copy(rows[b], acc.at[dst2_v.at[c]],
                              ssems[b]).wait()

    def scale(c, b):
        rb = rows[b]

        def scale_body(k, carry2):
            a16 = plsc.load_gather(ee_v, [_f16(c * _K + k, jnp.int32)])
            for j in range(_DH // 16):
                sl = pl.ds(j * 16, 16)
                rb[k, sl] = rb[k, sl] * a16
            return carry2

        lax.fori_loop(0, _K, scale_body, 0)

    start_gather(0, 0)
    start_gather(1, 1)
    for c in (0, 1):
        wait_gather(c, c)
        scale(c, c)
        start_scatter(c, c)
        start_gather(c + 2, c + 2)

    def steady(g, carry):
        for j in range(4):
            c = g * 4 + (2 + j)
            b = (2 + j) % 4
            b2 = j
            wait_gather(c, b)
            scale(c, b)
            start_scatter(c, b)
            wait_scatter(c - 2, b2)
            start_gather(c + 2, b2)
        return carry

    lax.fori_loop(0, (_NCH - 4) // 4, steady, 0)

    for c in (_NCH - 2, _NCH - 1):
        b = c % 4
        wait_gather(c, b)
        scale(c, b)
        start_scatter(c, b)
    for c in (_NCH - 4, _NCH - 3, _NCH - 2, _NCH - 1):
        wait_scatter(c, c % 4)

    plsc.subcore_barrier()

    def out_body(r, carry):
        sl = pl.ds(sid * _NROW + r * _ZCH, _ZCH)
        pltpu.sync_copy(acc.at[sl], out_hbm.at[cid].at[sl])
        return carry

    lax.fori_loop(0, _NROW // _ZCH, out_body, 0)


_sc_agg = functools.partial(
    pl.kernel,
    out_type=jax.ShapeDtypeStruct((_NC, _NP, _DH), jnp.float32),
    mesh=_MESH,
    compiler_params=_SC_PARAMS_NT,
    scratch_types=[
        pltpu.VMEM((_NCH, _K), jnp.int32),
        pltpu.VMEM((_NCH, _K), jnp.int32),
        pltpu.VMEM((_EPW,), jnp.float32),
        pltpu.VMEM((_K, _DH), jnp.float32),
        pltpu.VMEM((_K, _DH), jnp.float32),
        pltpu.VMEM((_K, _DH), jnp.float32),
        pltpu.VMEM((_K, _DH), jnp.float32),
        pltpu.VMEM_SHARED((_NP, _DH), jnp.float32),
        pltpu.SemaphoreType.DMA,
        pltpu.SemaphoreType.DMA,
        pltpu.SemaphoreType.DMA,
        pltpu.SemaphoreType.DMA,
        pltpu.SemaphoreType.DMA,
        pltpu.SemaphoreType.DMA,
        pltpu.SemaphoreType.DMA,
        pltpu.SemaphoreType.DMA,
    ],
)(_sc_agg_body)


def _layer(h, src, dst, src3, dst3, ea, W0, W1, W2, Wa):
    w1t = W1.T
    w2t = W2.T
    wa2 = jnp.stack([Wa[0, :_D], Wa[0, _D:2 * _D]], axis=1)
    q16 = jnp.full((16,), W0[0, 0] * Wa[0, 2 * _D], jnp.float32)

    z0, z1, zi, a2 = _tc_pre(h, w1t, w2t, wa2)
    asrc = a2[:, 0]
    adst = a2[:, 1]
    e, mpart = _sc_logit(src, dst, ea, asrc, adst, q16)
    m = _tc_max(mpart).reshape(_N)
    ee, spart = _sc_expsum(dst, e, m)
    invd = _tc_invsum(spart).reshape(_N, 1)
    pa = _sc_agg(src3, dst3, ee, z0)
    pb = _sc_agg(src3, dst3, ee, z1)
    return _tc_out(zi, pa[0, :_N], pa[1, :_N], pb[0, :_N], pb[1, :_N], invd)


def kernel(x, edge_index, edge_attr, W0_0, W1_0, W2_0, Wa_0,
           W0_1, W1_1, W2_1, Wa_1):
    src = edge_index[0]
    dst = edge_index[1]
    ea = edge_attr[:, 0]
    src3 = src.reshape(_NW, _NCH, _K)
    dst3 = dst.reshape(_NW, _NCH, _K)
    h = _layer(x, src, dst, src3, dst3, ea, W0_0, W1_0, W2_0, Wa_0)
    h = _layer(h, src, dst, src3, dst3, ea, W0_1, W1_1, W2_1, Wa_1)
    return h


# speedup vs baseline: 26.8488x; 1.2054x over previous
import functools

import jax
import jax.numpy as jnp
from jax import lax
from jax.experimental import pallas as pl
from jax.experimental.pallas import tpu as pltpu
from jax.experimental.pallas import tpu_sc as plsc

_N = 10000
_E = 320000
_D = 128
_DH = _D // 2
_NC = 2
_NS = 16
_NW = _NC * _NS
_EPW = _E // _NW
_K = 100
_NCH = _EPW // _K
_NP = 10240
_NROW = _NP // _NS
_ZCH = 80

_MESH = plsc.VectorSubcoreMesh(core_axis_name="c", subcore_axis_name="s")
_SC_PARAMS = pltpu.CompilerParams(needs_layout_passes=False)
_SC_PARAMS_NT = pltpu.CompilerParams(needs_layout_passes=False,
                                     use_tc_tiling_on_sc=False)


def _f16(val, dtype=jnp.float32):
    return jnp.full((16,), val, dtype)


_GDN = lax.GatherDimensionNumbers(
    offset_dims=(), collapsed_slice_dims=(0,), start_index_map=(0,))


def _vtake(vec, idx):
    return lax.gather(vec, idx[:, None], _GDN, (1,),
                      mode=lax.GatherScatterMode.PROMISE_IN_BOUNDS)


def _seg_last_and_prefix(keys, vals, op):
    iota = lax.iota(jnp.int32, 16)
    v = vals
    for step in (1, 2, 4, 8):
        idx = jnp.maximum(iota - step, 0)
        same = (_vtake(keys, idx) == keys) & (iota >= step)
        v = jnp.where(same, op(v, _vtake(v, idx)), v)
    nxt = _vtake(keys, jnp.minimum(iota + 1, 15))
    is_last = (keys != nxt) | (iota == 15)
    return v, is_last


_BN = 2000


def _tc_pre_body(h_ref, w1t_ref, w2t_ref, wa2_ref,
                 z0_ref, z1_ref, zi_ref, a2_ref):
    h = h_ref[...]
    z = jnp.dot(h, w1t_ref[...], preferred_element_type=jnp.float32)
    z0_ref[...] = z[:, :_DH]
    z1_ref[...] = z[:, _DH:]
    zi_ref[...] = jnp.dot(h, w2t_ref[...], preferred_element_type=jnp.float32)
    a2_ref[...] = jnp.dot(z, wa2_ref[...], preferred_element_type=jnp.float32)


def _tc_pre(h, w1t, w2t, wa2):
    return pl.pallas_call(
        _tc_pre_body,
        grid=(_N // _BN,),
        in_specs=[
            pl.BlockSpec((_BN, _D), lambda i: (i, 0)),
            pl.BlockSpec((_D, _D), lambda i: (0, 0)),
            pl.BlockSpec((_D, _D), lambda i: (0, 0)),
            pl.BlockSpec((_D, 2), lambda i: (0, 0)),
        ],
        out_specs=[
            pl.BlockSpec((_BN, _DH), lambda i: (i, 0)),
            pl.BlockSpec((_BN, _DH), lambda i: (i, 0)),
            pl.BlockSpec((_BN, _D), lambda i: (i, 0)),
            pl.BlockSpec((_BN, 2), lambda i: (i, 0)),
        ],
        out_shape=[
            jax.ShapeDtypeStruct((_N, _DH), jnp.float32),
            jax.ShapeDtypeStruct((_N, _DH), jnp.float32),
            jax.ShapeDtypeStruct((_N, _D), jnp.float32),
            jax.ShapeDtypeStruct((_N, 2), jnp.float32),
        ],
    )(h, w1t, w2t, wa2)


def _tc_max_body(p_ref, o_ref):
    o_ref[...] = jnp.max(p_ref[...], axis=0, keepdims=True)


def _tc_max(parts):
    return pl.pallas_call(
        _tc_max_body,
        out_shape=jax.ShapeDtypeStruct((1, _N), jnp.float32),
    )(parts)


def _tc_invsum_body(p_ref, o_ref):
    o_ref[...] = 1.0 / jnp.sum(p_ref[...], axis=0, keepdims=True)


def _tc_invsum(parts):
    return pl.pallas_call(
        _tc_invsum_body,
        out_shape=jax.ShapeDtypeStruct((1, _N), jnp.float32),
    )(parts)


def _tc_out_body(zi_ref, pa0_ref, pa1_ref, pb0_ref, pb1_ref, inv_ref, o_ref):
    zi = zi_ref[...]
    inv = inv_ref[...]
    left = zi[:, :_DH] + inv * (pa0_ref[0] + pa1_ref[0])
    right = zi[:, _DH:] + inv * (pb0_ref[0] + pb1_ref[0])
    o_ref[...] = jnp.maximum(jnp.concatenate([left, right], axis=1), 0.0)


def _tc_out(zi, pa, pb, inv_col):
    return pl.pallas_call(
        _tc_out_body,
        grid=(_N // _BN,),
        in_specs=[
            pl.BlockSpec((_BN, _D), lambda i: (i, 0)),
            pl.BlockSpec((1, _BN, _DH), lambda i: (0, i, 0)),
            pl.BlockSpec((1, _BN, _DH), lambda i: (1, i, 0)),
            pl.BlockSpec((1, _BN, _DH), lambda i: (0, i, 0)),
            pl.BlockSpec((1, _BN, _DH), lambda i: (1, i, 0)),
            pl.BlockSpec((_BN, 1), lambda i: (i, 0)),
        ],
        out_specs=pl.BlockSpec((_BN, _D), lambda i: (i, 0)),
        out_shape=jax.ShapeDtypeStruct((_N, _D), jnp.float32),
    )(zi, pa, pa, pb, pb, inv_col)


def _sc_logit_body(src_hbm, dst_hbm, ea_hbm, asrc_hbm, adst_hbm, q_hbm,
                   e_hbm, mpart_hbm,
                   asrc_v, adst_v, m_v, src_v, dst_v, ea_v, e_v, q_v):
    cid = lax.axis_index("c")
    sid = lax.axis_index("s")
    wid = sid * _NC + cid
    base = wid * _EPW
    pltpu.sync_copy(asrc_hbm, asrc_v)
    pltpu.sync_copy(adst_hbm, adst_v)
    pltpu.sync_copy(src_hbm.at[pl.ds(base, _EPW)], src_v)
    pltpu.sync_copy(dst_hbm.at[pl.ds(base, _EPW)], dst_v)
    pltpu.sync_copy(ea_hbm.at[pl.ds(base, _EPW)], ea_v)
    pltpu.sync_copy(q_hbm, q_v)

    neg_inf = _f16(-jnp.inf)

    def init_body(i, carry):
        for u in range(5):
            m_v[pl.ds((i * 5 + u) * 16, 16)] = neg_inf
        return carry

    lax.fori_loop(0, _N // 80, init_body, 0)

    q = q_v[...]

    def edge_body(i, carry):
        for u in range(5):
            sl = pl.ds((i * 5 + u) * 16, 16)
            s = src_v[sl]
            dvec = dst_v[sl]
            u_ = (plsc.load_gather(asrc_v, [s])
                  + plsc.load_gather(adst_v, [dvec]) + q * ea_v[sl])
            e = jnp.where(u_ >= 0.0, u_, u_ * 0.01)
            e_v[sl] = e
            k_s, v_s = plsc.sort_key_val(dvec, e)
            segmax, is_last = _seg_last_and_prefix(k_s, v_s, jnp.maximum)
            old = plsc.load_gather(m_v, [k_s])
            plsc.store_scatter(m_v, [k_s], jnp.maximum(old, segmax),
                               mask=is_last)
        return carry

    lax.fori_loop(0, _EPW // 80, edge_body, 0)

    pltpu.sync_copy(e_v, e_hbm.at[pl.ds(base, _EPW)])
    pltpu.sync_copy(m_v, mpart_hbm.at[wid])


_sc_logit = functools.partial(
    pl.kernel,
    out_type=[jax.ShapeDtypeStruct((_E,), jnp.float32),
              jax.ShapeDtypeStruct((_NW, _N), jnp.float32)],
    mesh=_MESH,
    compiler_params=_SC_PARAMS,
    scratch_types=[
        pltpu.VMEM((_N,), jnp.float32),
        pltpu.VMEM((_N,), jnp.float32),
        pltpu.VMEM((_N,), jnp.float32),
        pltpu.VMEM((_EPW,), jnp.int32),
        pltpu.VMEM((_EPW,), jnp.int32),
        pltpu.VMEM((_EPW,), jnp.float32),
        pltpu.VMEM((_EPW,), jnp.float32),
        pltpu.VMEM((16,), jnp.float32),
    ],
)(_sc_logit_body)


def _sc_expsum_body(dst_hbm, e_hbm, m_hbm,
                    ee_hbm, spart_hbm,
                    m_v, s_v, dst_v, e_v, ee_v):
    cid = lax.axis_index("c")
    sid = lax.axis_index("s")
    wid = sid * _NC + cid
    base = wid * _EPW
    pltpu.sync_copy(m_hbm, m_v)
    pltpu.sync_copy(dst_hbm.at[pl.ds(base, _EPW)], dst_v)
    pltpu.sync_copy(e_hbm.at[pl.ds(base, _EPW)], e_v)

    zero = _f16(0.0)

    def init_body(i, carry):
        for u in range(5):
            s_v[pl.ds((i * 5 + u) * 16, 16)] = zero
        return carry

    lax.fori_loop(0, _N // 80, init_body, 0)

    def edge_body(i, carry):
        for u in range(5):
            sl = pl.ds((i * 5 + u) * 16, 16)
            dvec = dst_v[sl]
            ee = jnp.exp(e_v[sl] - plsc.load_gather(m_v, [dvec]))
            ee_v[sl] = ee
            k_s, v_s = plsc.sort_key_val(dvec, ee)
            seg_tot, is_last = _seg_last_and_prefix(k_s, v_s, jnp.add)
            old = plsc.load_gather(s_v, [k_s])
            plsc.store_scatter(s_v, [k_s], old + seg_tot, mask=is_last)
        return carry

    lax.fori_loop(0, _EPW // 80, edge_body, 0)

    pltpu.sync_copy(ee_v, ee_hbm.at[pl.ds(base, _EPW)])
    pltpu.sync_copy(s_v, spart_hbm.at[wid])


_sc_expsum = functools.partial(
    pl.kernel,
    out_type=[jax.ShapeDtypeStruct((_E,), jnp.float32),
              jax.ShapeDtypeStruct((_NW, _N), jnp.float32)],
    mesh=_MESH,
    compiler_params=_SC_PARAMS,
    scratch_types=[
        pltpu.VMEM((_N,), jnp.float32),
        pltpu.VMEM((_N,), jnp.float32),
        pltpu.VMEM((_EPW,), jnp.int32),
        pltpu.VMEM((_EPW,), jnp.float32),
        pltpu.VMEM((_EPW,), jnp.float32),
    ],
)(_sc_expsum_body)


def _sc_agg_body(src3_hbm, dst3_hbm, ee_hbm, z_hbm,
                 out_hbm,
                 src2_v, dst2_v, ee_v, r0, r1, r2, r3, acc,
                 g0, g1, g2, g3, s0, s1, s2, s3):
    rows = (r0, r1, r2, r3)
    gsems = (g0, g1, g2, g3)
    ssems = (s0, s1, s2, s3)
    cid = lax.axis_index("c")
    sid = lax.axis_index("s")
    wid = sid * _NC + cid
    pltpu.sync_copy(src3_hbm.at[wid], src2_v)
    pltpu.sync_copy(dst3_hbm.at[wid], dst2_v)
    pltpu.sync_copy(ee_hbm.at[pl.ds(wid * _EPW, _EPW)], ee_v)

    for r in range(_ZCH):
        for j in range(_DH // 16):
            r0[r, pl.ds(j * 16, 16)] = _f16(0.0)

    def zero_body(r, carry):
        pltpu.sync_copy(r0.at[pl.ds(0, _ZCH)],
                        acc.at[pl.ds(sid * _NROW + r * _ZCH, _ZCH)])
        return carry

    lax.fori_loop(0, _NROW // _ZCH, zero_body, 0)
    plsc.subcore_barrier()

    def start_gather(c, b):
        pltpu.async_copy(z_hbm.at[src2_v.at[c]], rows[b], gsems[b])

    def wait_gather(c, b):
        pltpu.make_async_copy(z_hbm.at[src2_v.at[c]], rows[b],
                              gsems[b]).wait()

    def start_scatter(c, b):
        pltpu.async_copy(rows[b], acc.at[dst2_v.at[c]], ssems[b], add=True)

    def wait_scatter(c, b):
        pltpu.make_async_copy(rows[b], acc.at[dst2_v.at[c]],
                              ssems[b]).wait()

    def scale(c, b):
        rb = rows[b]

        def scale_body(k):
            a16 = plsc.load_gather(ee_v, [_f16(c * _K + k, jnp.int32)])
            for j in range(_DH // 16):
                sl = pl.ds(j * 16, 16)
                rb[k, sl] = rb[k, sl] * a16

        plsc.parallel_loop(0, _K, 1, unroll=4)(scale_body)

    start_gather(0, 0)
    start_gather(1, 1)
    for c in (0, 1):
        wait_gather(c, c)
        scale(c, c)
        start_scatter(c, c)
        start_gather(c + 2, c + 2)

    def steady(g, carry):
        for j in range(4):
            c = g * 4 + (2 + j)
            b = (2 + j) % 4
            b2 = j
            wait_gather(c, b)
            scale(c, b)
            start_scatter(c, b)
            wait_scatter(c - 2, b2)
            start_gather(c + 2, b2)
        return carry

    lax.fori_loop(0, (_NCH - 4) // 4, steady, 0)

    for c in (_NCH - 2, _NCH - 1):
        b = c % 4
        wait_gather(c, b)
        scale(c, b)
        start_scatter(c, b)
    for c in (_NCH - 4, _NCH - 3, _NCH - 2, _NCH - 1):
        wait_scatter(c, c % 4)

    plsc.subcore_barrier()

    def out_body(r, carry):
        sl = pl.ds(sid * _NROW + r * _ZCH, _ZCH)
        pltpu.sync_copy(acc.at[sl], out_hbm.at[cid].at[sl])
        return carry

    lax.fori_loop(0, _NROW // _ZCH, out_body, 0)


_sc_agg = functools.partial(
    pl.kernel,
    out_type=jax.ShapeDtypeStruct((_NC, _NP, _DH), jnp.float32),
    mesh=_MESH,
    compiler_params=_SC_PARAMS_NT,
    scratch_types=[
        pltpu.VMEM((_NCH, _K), jnp.int32),
        pltpu.VMEM((_NCH, _K), jnp.int32),
        pltpu.VMEM((_EPW,), jnp.float32),
        pltpu.VMEM((_K, _DH), jnp.float32),
        pltpu.VMEM((_K, _DH), jnp.float32),
        pltpu.VMEM((_K, _DH), jnp.float32),
        pltpu.VMEM((_K, _DH), jnp.float32),
        pltpu.VMEM_SHARED((_NP, _DH), jnp.float32),
        pltpu.SemaphoreType.DMA,
        pltpu.SemaphoreType.DMA,
        pltpu.SemaphoreType.DMA,
        pltpu.SemaphoreType.DMA,
        pltpu.SemaphoreType.DMA,
        pltpu.SemaphoreType.DMA,
        pltpu.SemaphoreType.DMA,
        pltpu.SemaphoreType.DMA,
    ],
)(_sc_agg_body)


def _layer(h, src, dst, src3, dst3, ea, W0, W1, W2, Wa):
    w1t = W1.T
    w2t = W2.T
    wa2 = jnp.stack([Wa[0, :_D], Wa[0, _D:2 * _D]], axis=1)
    q16 = jnp.full((16,), W0[0, 0] * Wa[0, 2 * _D], jnp.float32)

    z0, z1, zi, a2 = _tc_pre(h, w1t, w2t, wa2)
    asrc = a2[:, 0]
    adst = a2[:, 1]
    e, mpart = _sc_logit(src, dst, ea, asrc, adst, q16)
    m = _tc_max(mpart).reshape(_N)
    ee, spart = _sc_expsum(dst, e, m)
    invd = _tc_invsum(spart).reshape(_N, 1)
    pa = _sc_agg(src3, dst3, ee, z0)
    pb = _sc_agg(src3, dst3, ee, z1)
    return _tc_out(zi, pa, pb, invd)


def kernel(x, edge_index, edge_attr, W0_0, W1_0, W2_0, Wa_0,
           W0_1, W1_1, W2_1, Wa_1):
    src = edge_index[0]
    dst = edge_index[1]
    ea = edge_attr[:, 0]
    src3 = src.reshape(_NW, _NCH, _K)
    dst3 = dst.reshape(_NW, _NCH, _K)
    h = _layer(x, src, dst, src3, dst3, ea, W0_0, W1_0, W2_0, Wa_0)
    h = _layer(h, src, dst, src3, dst3, ea, W0_1, W1_1, W2_1, Wa_1)
    return h
